# baseline (speedup 1.0000x reference)
import functools

import jax
import jax.numpy as jnp
from jax import lax
from jax.experimental import pallas as pl
from jax.experimental.pallas import tpu as pltpu
from jax.experimental.pallas import tpu_sc as plsc

N = 10000
E = 320000
NF = 128
HC = 128
NCLS = 64

SC_CORES = 2
SC_TILES = 16
NW = SC_CORES * SC_TILES

NPAD = 10240
ROWS_PER_TILE = NPAD // SC_TILES
EPC = E // SC_CORES
EPT = EPC // SC_TILES
CHUNK_A = 50
NCHUNK_A = EPT // CHUNK_A
CHUNK_B = 125
NCHUNK_B = EPT // CHUNK_B

_mesh = plsc.VectorSubcoreMesh(core_axis_name="c", subcore_axis_name="s")


def _seg_body(d, with_cnt, NBUF, IDEPTH, NCHUNK, x_hbm, es_hbm, zeros_hbm,
              zerosc_hbm, ones_hbm, out_hbm, outc_hbm, ib_v, rows_v, ones_v,
              acc_sh, cnt_sh, isem, gsem, ssem, csem, zsem):
    cid = lax.axis_index("c")
    sid = lax.axis_index("s")
    row0 = sid * ROWS_PER_TILE

    for k in range(IDEPTH):
        pltpu.async_copy(es_hbm.at[cid, sid, k], ib_v.at[k], isem.at[k])
    pltpu.async_copy(zeros_hbm, acc_sh.at[pl.ds(row0, ROWS_PER_TILE)], zsem)
    if with_cnt:
        pltpu.async_copy(zerosc_hbm, cnt_sh.at[pl.ds(row0, ROWS_PER_TILE)], zsem)
        pltpu.async_copy(ones_hbm, ones_v, zsem)
    for k in range(NBUF - 1):
        pltpu.make_async_copy(es_hbm.at[cid, sid, k], ib_v.at[k],
                              isem.at[k]).wait()
        pltpu.async_copy(x_hbm.at[ib_v.at[k, 0]], rows_v.at[k], gsem.at[k])
    pltpu.make_async_copy(zeros_hbm, acc_sh.at[pl.ds(row0, ROWS_PER_TILE)],
                          zsem).wait()
    if with_cnt:
        pltpu.make_async_copy(zerosc_hbm, cnt_sh.at[pl.ds(row0, ROWS_PER_TILE)],
                              zsem).wait()
        pltpu.make_async_copy(ones_hbm, ones_v, zsem).wait()
    plsc.subcore_barrier()

    def body(j, carry):
        b = lax.rem(j, NBUF)
        t = lax.rem(j, IDEPTH)
        pltpu.make_async_copy(x_hbm.at[ib_v.at[t, 0]], rows_v.at[b],
                              gsem.at[b]).wait()
        pltpu.async_copy(rows_v.at[b], acc_sh.at[ib_v.at[t, 1]], ssem.at[b],
                         add=True)
        if with_cnt:
            pltpu.async_copy(ones_v, cnt_sh.at[ib_v.at[t, 1]], csem.at[b],
                             add=True)

        @pl.when(j + NBUF - 1 < NCHUNK)
        def _():
            bp = lax.rem(j + NBUF - 1, NBUF)
            tp = lax.rem(j + IDEPTH - 1, IDEPTH)

            @pl.when(j >= 1)
            def _():
                pltpu.make_async_copy(rows_v.at[bp], acc_sh.at[ib_v.at[tp, 1]],
                                      ssem.at[bp]).wait()
                if with_cnt:
                    pltpu.make_async_copy(ones_v, cnt_sh.at[ib_v.at[tp, 1]],
                                          csem.at[bp]).wait()

                @pl.when(j - 1 + IDEPTH < NCHUNK)
                def _():
                    pltpu.async_copy(es_hbm.at[cid, sid, j - 1 + IDEPTH],
                                     ib_v.at[tp], isem.at[tp])

            tk = lax.rem(j + NBUF - 1, IDEPTH)
            pltpu.make_async_copy(es_hbm.at[cid, sid, j + NBUF - 1],
                                  ib_v.at[tk], isem.at[tk]).wait()
            pltpu.async_copy(x_hbm.at[ib_v.at[tk, 0]], rows_v.at[bp],
                             gsem.at[bp])

        return carry

    lax.fori_loop(0, NCHUNK, body, 0)
    for k in range(NCHUNK - NBUF, NCHUNK):
        _b = k % NBUF
        _t = k % IDEPTH
        pltpu.make_async_copy(rows_v.at[_b], acc_sh.at[ib_v.at[_t, 1]],
                              ssem.at[_b]).wait()
        if with_cnt:
            pltpu.make_async_copy(ones_v, cnt_sh.at[ib_v.at[_t, 1]],
                                  csem.at[_b]).wait()
    plsc.subcore_barrier()

    pltpu.async_copy(acc_sh.at[pl.ds(row0, ROWS_PER_TILE)],
                     out_hbm.at[cid, pl.ds(row0, ROWS_PER_TILE)], zsem)
    if with_cnt:
        pltpu.async_copy(cnt_sh.at[pl.ds(row0, ROWS_PER_TILE)],
                         outc_hbm.at[cid, pl.ds(row0, ROWS_PER_TILE)], zsem)
        pltpu.make_async_copy(cnt_sh.at[pl.ds(row0, ROWS_PER_TILE)],
                              outc_hbm.at[cid, pl.ds(row0, ROWS_PER_TILE)],
                              zsem).wait()
    pltpu.make_async_copy(acc_sh.at[pl.ds(row0, ROWS_PER_TILE)],
                          out_hbm.at[cid, pl.ds(row0, ROWS_PER_TILE)],
                          zsem).wait()


def _make_seg_kernel(d, with_cnt, NBUF, IDEPTH, CHUNK):
    NCHUNK = EPT // CHUNK
    if with_cnt:
        out_type = [jax.ShapeDtypeStruct((SC_CORES, NPAD, d), jnp.float32),
                    jax.ShapeDtypeStruct((SC_CORES, NPAD, 16), jnp.float32)]
    else:
        out_type = jax.ShapeDtypeStruct((SC_CORES, NPAD, d), jnp.float32)
    scratch = [
        pltpu.VMEM((IDEPTH, 2, CHUNK), jnp.int32),
        pltpu.VMEM((NBUF, CHUNK, d), jnp.float32),
        pltpu.VMEM((CHUNK, 16), jnp.float32),
        pltpu.VMEM_SHARED((NPAD, d), jnp.float32),
        pltpu.VMEM_SHARED((NPAD, 16), jnp.float32) if with_cnt else None,
        pltpu.SemaphoreType.DMA((IDEPTH,)),
        pltpu.SemaphoreType.DMA((NBUF,)),
        pltpu.SemaphoreType.DMA((NBUF,)),
        pltpu.SemaphoreType.DMA((NBUF,)) if with_cnt else None,
        pltpu.SemaphoreType.DMA,
    ]
    scratch = [s for s in scratch if s is not None]

    if with_cnt:
        def body(x_hbm, es_hbm, zeros_hbm, zerosc_hbm, ones_hbm,
                 out_hbm, outc_hbm, ib_v, rows_v, ones_v, acc_sh,
                 cnt_sh, isem, gsem, ssem, csem, zsem):
            _seg_body(d, True, NBUF, IDEPTH, NCHUNK, x_hbm, es_hbm, zeros_hbm,
                      zerosc_hbm, ones_hbm, out_hbm, outc_hbm, ib_v, rows_v,
                      ones_v, acc_sh, cnt_sh, isem, gsem, ssem, csem, zsem)
    else:
        def body(x_hbm, es_hbm, zeros_hbm, zerosc_hbm, ones_hbm,
                 out_hbm, ib_v, rows_v, ones_v, acc_sh, isem, gsem, ssem,
                 zsem):
            _seg_body(d, False, NBUF, IDEPTH, NCHUNK, x_hbm, es_hbm,
                      zeros_hbm, zerosc_hbm, ones_hbm, out_hbm, None, ib_v,
                      rows_v, ones_v, acc_sh, None, isem, gsem, ssem, None,
                      zsem)

    return pl.kernel(body, out_type=out_type, mesh=_mesh,
                     scratch_types=scratch, name=f"sage_seg_{d}",
                     compiler_params=pltpu.CompilerParams(
                         use_tc_tiling_on_sc=False))


_seg128 = _make_seg_kernel(NF, True, 4, 6, CHUNK_A)
_seg64 = _make_seg_kernel(NCLS, False, 6, 8, CHUNK_B)

R = 400


def _tc0_body(x_ref, wr1_ref, b1_ref, xr_ref):
    xr_ref[...] = (jnp.dot(x_ref[...], wr1_ref[...],
                           preferred_element_type=jnp.float32) + b1_ref[...])


def _tc1_body(pa_ref, pb_ref, ca_ref, cb_ref, xr_ref, wl1_ref,
              wl2_ref, emb_ref, hw_ref):
    cnt = ca_ref[0][:, 0:1] + cb_ref[0][:, 0:1]
    mean = (pa_ref[0] + pb_ref[0]) / jnp.maximum(cnt, 1.0)
    emb = (jnp.dot(mean, wl1_ref[...], preferred_element_type=jnp.float32)
           + xr_ref[...])
    h = jnp.maximum(emb, 0.0)
    emb_ref[...] = emb
    hw_ref[...] = jnp.dot(h, wl2_ref[...], preferred_element_type=jnp.float32)


def _tc1b_body(emb_ref, wr2_ref, b2_ref, hr_ref):
    h = jnp.maximum(emb_ref[...], 0.0)
    hr_ref[...] = (jnp.dot(h, wr2_ref[...], preferred_element_type=jnp.float32)
                   + b2_ref[...])


def _tc2_body(pa_ref, pb_ref, ca_ref, cb_ref, hr_ref, out_ref):
    cnt = ca_ref[0][:, 0:1] + cb_ref[0][:, 0:1]
    out_ref[...] = (pa_ref[0] + pb_ref[0]) / jnp.maximum(cnt, 1.0) + hr_ref[...]


def _part_spec(core, d):
    return pl.BlockSpec((1, R, d), lambda i, c=core: (c, i, 0))


_tc0 = pl.pallas_call(
    _tc0_body,
    grid=(N // R,),
    in_specs=[
        pl.BlockSpec((R, NF), lambda i: (i, 0)),
        pl.BlockSpec((NF, HC), lambda i: (0, 0)),
        pl.BlockSpec((1, HC), lambda i: (0, 0)),
    ],
    out_specs=pl.BlockSpec((R, HC), lambda i: (i, 0)),
    out_shape=jax.ShapeDtypeStruct((N, HC), jnp.float32),
)

_tc1 = pl.pallas_call(
    _tc1_body,
    grid=(N // R,),
    in_specs=[
        _part_spec(0, NF), _part_spec(1, NF),
        _part_spec(0, 16), _part_spec(1, 16),
        pl.BlockSpec((R, HC), lambda i: (i, 0)),
        pl.BlockSpec((NF, HC), lambda i: (0, 0)),
        pl.BlockSpec((HC, NCLS), lambda i: (0, 0)),
    ],
    out_specs=[
        pl.BlockSpec((R, HC), lambda i: (i, 0)),
        pl.BlockSpec((R, NCLS), lambda i: (i, 0)),
    ],
    out_shape=[
        jax.ShapeDtypeStruct((N, HC), jnp.float32),
        jax.ShapeDtypeStruct((N, NCLS), jnp.float32),
    ],
)

_tc1b = pl.pallas_call(
    _tc1b_body,
    grid=(N // R,),
    in_specs=[
        pl.BlockSpec((R, HC), lambda i: (i, 0)),
        pl.BlockSpec((HC, NCLS), lambda i: (0, 0)),
        pl.BlockSpec((1, NCLS), lambda i: (0, 0)),
    ],
    out_specs=pl.BlockSpec((R, NCLS), lambda i: (i, 0)),
    out_shape=jax.ShapeDtypeStruct((N, NCLS), jnp.float32),
)

_tc2 = pl.pallas_call(
    _tc2_body,
    grid=(N // R,),
    in_specs=[
        _part_spec(0, NCLS), _part_spec(1, NCLS),
        _part_spec(0, 16), _part_spec(1, 16),
        pl.BlockSpec((R, NCLS), lambda i: (i, 0)),
    ],
    out_specs=pl.BlockSpec((R, NCLS), lambda i: (i, 0)),
    out_shape=jax.ShapeDtypeStruct((N, NCLS), jnp.float32),
)

_ZEROS = None
_ONES = None


def kernel(x, edge_index, W_l1, W_r1, b1, W_l2, W_r2, b2):
    src = edge_index[0].astype(jnp.int32)
    dst = edge_index[1].astype(jnp.int32)
    es_a = jnp.stack(
        [src.reshape(SC_CORES, SC_TILES, NCHUNK_A, CHUNK_A),
         dst.reshape(SC_CORES, SC_TILES, NCHUNK_A, CHUNK_A)], axis=3)
    es_b = jnp.stack(
        [src.reshape(SC_CORES, SC_TILES, NCHUNK_B, CHUNK_B),
         dst.reshape(SC_CORES, SC_TILES, NCHUNK_B, CHUNK_B)], axis=3)
    zeros128 = jnp.zeros((ROWS_PER_TILE, NF), jnp.float32)
    zeros64 = jnp.zeros((ROWS_PER_TILE, NCLS), jnp.float32)
    zeros16 = jnp.zeros((ROWS_PER_TILE, 16), jnp.float32)
    ones = jnp.ones((CHUNK_A, 16), jnp.float32)

    p1, c1 = _seg128(x, es_a, zeros128, zeros16, ones)
    xr = _tc0(x, W_r1, b1.reshape(1, HC))
    emb, hw = _tc1(p1, p1, c1, c1, xr, W_l1, W_l2)
    p2 = _seg64(hw, es_b, zeros64, zeros16, ones)
    hr = _tc1b(emb, W_r2, b2.reshape(1, NCLS))
    logits = _tc2(p2, p2, c1, c1, hr)
    return (logits, emb)


# speedup vs baseline: 1.1021x; 1.1021x over previous
import functools

import jax
import jax.numpy as jnp
from jax import lax
from jax.experimental import pallas as pl
from jax.experimental.pallas import tpu as pltpu
from jax.experimental.pallas import tpu_sc as plsc

N = 10000
E = 320000
NF = 128
HC = 128
NCLS = 64

SC_CORES = 2
SC_TILES = 16
NW = SC_CORES * SC_TILES

NPAD = 10240
ROWS_PER_TILE = NPAD // SC_TILES
EPC = E // SC_CORES
EPT = EPC // SC_TILES
CHUNK_A = 80
NCHUNK_A = EPT // CHUNK_A
CHUNK_B = 125
NCHUNK_B = EPT // CHUNK_B

_mesh = plsc.VectorSubcoreMesh(core_axis_name="c", subcore_axis_name="s")


def _seg_body(d, with_cnt, NBUF, IDEPTH, NCHUNK, x_hbm, es_hbm, zeros_hbm,
              zerosc_hbm, ones_hbm, out_hbm, outc_hbm, ib_v, rows_v, ones_v,
              acc_sh, cnt_sh, isem, gsem, ssem, csem, zsem):
    cid = lax.axis_index("c")
    sid = lax.axis_index("s")
    row0 = sid * ROWS_PER_TILE

    for k in range(IDEPTH):
        pltpu.async_copy(es_hbm.at[cid, sid, k], ib_v.at[k], isem.at[k])
    pltpu.async_copy(zeros_hbm, acc_sh.at[pl.ds(row0, ROWS_PER_TILE)], zsem)
    if with_cnt:
        pltpu.async_copy(zerosc_hbm, cnt_sh.at[pl.ds(row0, ROWS_PER_TILE)], zsem)
        pltpu.async_copy(ones_hbm, ones_v, zsem)
    for k in range(NBUF - 1):
        pltpu.make_async_copy(es_hbm.at[cid, sid, k], ib_v.at[k],
                              isem.at[k]).wait()
        pltpu.async_copy(x_hbm.at[ib_v.at[k, 0]], rows_v.at[k], gsem.at[k])
    pltpu.make_async_copy(zeros_hbm, acc_sh.at[pl.ds(row0, ROWS_PER_TILE)],
                          zsem).wait()
    if with_cnt:
        pltpu.make_async_copy(zerosc_hbm, cnt_sh.at[pl.ds(row0, ROWS_PER_TILE)],
                              zsem).wait()
        pltpu.make_async_copy(ones_hbm, ones_v, zsem).wait()
    plsc.subcore_barrier()

    def body(j, carry):
        b = lax.rem(j, NBUF)
        t = lax.rem(j, IDEPTH)
        pltpu.make_async_copy(x_hbm.at[ib_v.at[t, 0]], rows_v.at[b],
                              gsem.at[b]).wait()
        pltpu.async_copy(rows_v.at[b], acc_sh.at[ib_v.at[t, 1]], ssem.at[b],
                         add=True)
        if with_cnt:
            pltpu.async_copy(ones_v, cnt_sh.at[ib_v.at[t, 1]], csem.at[b],
                             add=True)

        @pl.when(j + NBUF - 1 < NCHUNK)
        def _():
            bp = lax.rem(j + NBUF - 1, NBUF)
            tp = lax.rem(j + IDEPTH - 1, IDEPTH)

            @pl.when(j >= 1)
            def _():
                pltpu.make_async_copy(rows_v.at[bp], acc_sh.at[ib_v.at[tp, 1]],
                                      ssem.at[bp]).wait()
                if with_cnt:
                    pltpu.make_async_copy(ones_v, cnt_sh.at[ib_v.at[tp, 1]],
                                          csem.at[bp]).wait()

                @pl.when(j - 1 + IDEPTH < NCHUNK)
                def _():
                    pltpu.async_copy(es_hbm.at[cid, sid, j - 1 + IDEPTH],
                                     ib_v.at[tp], isem.at[tp])

            tk = lax.rem(j + NBUF - 1, IDEPTH)
            pltpu.make_async_copy(es_hbm.at[cid, sid, j + NBUF - 1],
                                  ib_v.at[tk], isem.at[tk]).wait()
            pltpu.async_copy(x_hbm.at[ib_v.at[tk, 0]], rows_v.at[bp],
                             gsem.at[bp])

        return carry

    lax.fori_loop(0, NCHUNK, body, 0)
    for k in range(NCHUNK - NBUF, NCHUNK):
        _b = k % NBUF
        _t = k % IDEPTH
        pltpu.make_async_copy(rows_v.at[_b], acc_sh.at[ib_v.at[_t, 1]],
                              ssem.at[_b]).wait()
        if with_cnt:
            pltpu.make_async_copy(ones_v, cnt_sh.at[ib_v.at[_t, 1]],
                                  csem.at[_b]).wait()
    plsc.subcore_barrier()

    pltpu.async_copy(acc_sh.at[pl.ds(row0, ROWS_PER_TILE)],
                     out_hbm.at[cid, pl.ds(row0, ROWS_PER_TILE)], zsem)
    if with_cnt:
        pltpu.async_copy(cnt_sh.at[pl.ds(row0, ROWS_PER_TILE)],
                         outc_hbm.at[cid, pl.ds(row0, ROWS_PER_TILE)], zsem)
        pltpu.make_async_copy(cnt_sh.at[pl.ds(row0, ROWS_PER_TILE)],
                              outc_hbm.at[cid, pl.ds(row0, ROWS_PER_TILE)],
                              zsem).wait()
    pltpu.make_async_copy(acc_sh.at[pl.ds(row0, ROWS_PER_TILE)],
                          out_hbm.at[cid, pl.ds(row0, ROWS_PER_TILE)],
                          zsem).wait()


def _make_seg_kernel(d, with_cnt, NBUF, IDEPTH, CHUNK):
    NCHUNK = EPT // CHUNK
    if with_cnt:
        out_type = [jax.ShapeDtypeStruct((SC_CORES, NPAD, d), jnp.float32),
                    jax.ShapeDtypeStruct((SC_CORES, NPAD, 8), jnp.float32)]
    else:
        out_type = jax.ShapeDtypeStruct((SC_CORES, NPAD, d), jnp.float32)
    scratch = [
        pltpu.VMEM((IDEPTH, 2, CHUNK), jnp.int32),
        pltpu.VMEM((NBUF, CHUNK, d), jnp.float32),
        pltpu.VMEM((CHUNK, 8), jnp.float32),
        pltpu.VMEM_SHARED((NPAD, d), jnp.float32),
        pltpu.VMEM_SHARED((NPAD, 8), jnp.float32) if with_cnt else None,
        pltpu.SemaphoreType.DMA((IDEPTH,)),
        pltpu.SemaphoreType.DMA((NBUF,)),
        pltpu.SemaphoreType.DMA((NBUF,)),
        pltpu.SemaphoreType.DMA((NBUF,)) if with_cnt else None,
        pltpu.SemaphoreType.DMA,
    ]
    scratch = [s for s in scratch if s is not None]

    if with_cnt:
        def body(x_hbm, es_hbm, zeros_hbm, zerosc_hbm, ones_hbm,
                 out_hbm, outc_hbm, ib_v, rows_v, ones_v, acc_sh,
                 cnt_sh, isem, gsem, ssem, csem, zsem):
            _seg_body(d, True, NBUF, IDEPTH, NCHUNK, x_hbm, es_hbm, zeros_hbm,
                      zerosc_hbm, ones_hbm, out_hbm, outc_hbm, ib_v, rows_v,
                      ones_v, acc_sh, cnt_sh, isem, gsem, ssem, csem, zsem)
    else:
        def body(x_hbm, es_hbm, zeros_hbm, zerosc_hbm, ones_hbm,
                 out_hbm, ib_v, rows_v, ones_v, acc_sh, isem, gsem, ssem,
                 zsem):
            _seg_body(d, False, NBUF, IDEPTH, NCHUNK, x_hbm, es_hbm,
                      zeros_hbm, zerosc_hbm, ones_hbm, out_hbm, None, ib_v,
                      rows_v, ones_v, acc_sh, None, isem, gsem, ssem, None,
                      zsem)

    return pl.kernel(body, out_type=out_type, mesh=_mesh,
                     scratch_types=scratch, name=f"sage_seg_{d}",
                     compiler_params=pltpu.CompilerParams(
                         use_tc_tiling_on_sc=False))


_seg128 = _make_seg_kernel(NF, True, 4, 6, CHUNK_A)
_seg64 = _make_seg_kernel(NCLS, False, 6, 8, CHUNK_B)

R = 400


def _tc0_body(x_ref, wr1_ref, b1_ref, xr_ref):
    xr_ref[...] = (jnp.dot(x_ref[...], wr1_ref[...],
                           preferred_element_type=jnp.float32) + b1_ref[...])


def _tc1_body(pa_ref, pb_ref, ca_ref, cb_ref, xr_ref, wl1_ref,
              wl2_ref, emb_ref, hw_ref):
    cnt = ca_ref[0][:, 0:1] + cb_ref[0][:, 0:1]
    mean = (pa_ref[0] + pb_ref[0]) / jnp.maximum(cnt, 1.0)
    emb = (jnp.dot(mean, wl1_ref[...], preferred_element_type=jnp.float32)
           + xr_ref[...])
    h = jnp.maximum(emb, 0.0)
    emb_ref[...] = emb
    hw_ref[...] = jnp.dot(h, wl2_ref[...], preferred_element_type=jnp.float32)


def _tc1b_body(emb_ref, wr2_ref, b2_ref, hr_ref):
    h = jnp.maximum(emb_ref[...], 0.0)
    hr_ref[...] = (jnp.dot(h, wr2_ref[...], preferred_element_type=jnp.float32)
                   + b2_ref[...])


def _tc2_body(pa_ref, pb_ref, ca_ref, cb_ref, hr_ref, out_ref):
    cnt = ca_ref[0][:, 0:1] + cb_ref[0][:, 0:1]
    out_ref[...] = (pa_ref[0] + pb_ref[0]) / jnp.maximum(cnt, 1.0) + hr_ref[...]


def _part_spec(core, d):
    return pl.BlockSpec((1, R, d), lambda i, c=core: (c, i, 0))


_tc0 = pl.pallas_call(
    _tc0_body,
    grid=(N // R,),
    in_specs=[
        pl.BlockSpec((R, NF), lambda i: (i, 0)),
        pl.BlockSpec((NF, HC), lambda i: (0, 0)),
        pl.BlockSpec((1, HC), lambda i: (0, 0)),
    ],
    out_specs=pl.BlockSpec((R, HC), lambda i: (i, 0)),
    out_shape=jax.ShapeDtypeStruct((N, HC), jnp.float32),
)

_tc1 = pl.pallas_call(
    _tc1_body,
    grid=(N // R,),
    in_specs=[
        _part_spec(0, NF), _part_spec(1, NF),
        _part_spec(0, 8), _part_spec(1, 8),
        pl.BlockSpec((R, HC), lambda i: (i, 0)),
        pl.BlockSpec((NF, HC), lambda i: (0, 0)),
        pl.BlockSpec((HC, NCLS), lambda i: (0, 0)),
    ],
    out_specs=[
        pl.BlockSpec((R, HC), lambda i: (i, 0)),
        pl.BlockSpec((R, NCLS), lambda i: (i, 0)),
    ],
    out_shape=[
        jax.ShapeDtypeStruct((N, HC), jnp.float32),
        jax.ShapeDtypeStruct((N, NCLS), jnp.float32),
    ],
)

_tc1b = pl.pallas_call(
    _tc1b_body,
    grid=(N // R,),
    in_specs=[
        pl.BlockSpec((R, HC), lambda i: (i, 0)),
        pl.BlockSpec((HC, NCLS), lambda i: (0, 0)),
        pl.BlockSpec((1, NCLS), lambda i: (0, 0)),
    ],
    out_specs=pl.BlockSpec((R, NCLS), lambda i: (i, 0)),
    out_shape=jax.ShapeDtypeStruct((N, NCLS), jnp.float32),
)

_tc2 = pl.pallas_call(
    _tc2_body,
    grid=(N // R,),
    in_specs=[
        _part_spec(0, NCLS), _part_spec(1, NCLS),
        _part_spec(0, 8), _part_spec(1, 8),
        pl.BlockSpec((R, NCLS), lambda i: (i, 0)),
    ],
    out_specs=pl.BlockSpec((R, NCLS), lambda i: (i, 0)),
    out_shape=jax.ShapeDtypeStruct((N, NCLS), jnp.float32),
)

_ZEROS = None
_ONES = None


def kernel(x, edge_index, W_l1, W_r1, b1, W_l2, W_r2, b2):
    src = edge_index[0].astype(jnp.int32)
    dst = edge_index[1].astype(jnp.int32)
    es_a = jnp.stack(
        [src.reshape(SC_CORES, SC_TILES, NCHUNK_A, CHUNK_A),
         dst.reshape(SC_CORES, SC_TILES, NCHUNK_A, CHUNK_A)], axis=3)
    es_b = jnp.stack(
        [src.reshape(SC_CORES, SC_TILES, NCHUNK_B, CHUNK_B),
         dst.reshape(SC_CORES, SC_TILES, NCHUNK_B, CHUNK_B)], axis=3)
    zeros128 = jnp.zeros((ROWS_PER_TILE, NF), jnp.float32)
    zeros64 = jnp.zeros((ROWS_PER_TILE, NCLS), jnp.float32)
    zeros8 = jnp.zeros((ROWS_PER_TILE, 8), jnp.float32)
    ones = jnp.ones((CHUNK_A, 8), jnp.float32)

    p1, c1 = _seg128(x, es_a, zeros128, zeros8, ones)
    xr = _tc0(x, W_r1, b1.reshape(1, HC))
    emb, hw = _tc1(p1, p1, c1, c1, xr, W_l1, W_l2)
    p2 = _seg64(hw, es_b, zeros64, zeros8, ones)
    hr = _tc1b(emb, W_r2, b2.reshape(1, NCLS))
    logits = _tc2(p2, p2, c1, c1, hr)
    return (logits, emb)
